# baseline (speedup 1.0000x reference)
import jax
import jax.numpy as jnp

@jax.jit
def kernel(t, Ws0, bs0, Wt0, bt0, Ws1, bs1, Wt1, bt1):
    return jnp.maximum(t @ Ws0.T + bs0, 0.0)
